# baseline (speedup 1.0000x reference)
import functools

import jax
import jax.numpy as jnp
from jax import lax
from jax.experimental import pallas as pl
from jax.experimental.pallas import tpu as pltpu
from jax.experimental.pallas import tpu_sc as plsc

NC = 2
NS = 16
LANES = 16
NW = NC * NS

CHUNK = 400
GW = 80


def _body(L, D, ids_hbm, unfair_hbm, fair_hbm, pos_hbm, tm_hbm,
          out_hbm,
          ids_s0, ids_s1, fid_s0, fid_s1, buf0, buf1,
          stf_v, stp_v, pidx_v, cnt_v,
          sem_in0, sem_in1, sem_out0, sem_out1,
          sem_ids0, sem_ids1, sem_g0, sem_g1, sem_fix):
    ids_s = (ids_s0, ids_s1)
    fid_s = (fid_s0, fid_s1)
    buf_v = (buf0, buf1)
    sem_in = (sem_in0, sem_in1)
    sem_out = (sem_out0, sem_out1)
    sem_ids = (sem_ids0, sem_ids1)
    sem_g = (sem_g0, sem_g1)

    wid = lax.axis_index("s") * NC + lax.axis_index("c")
    per_w = ids_hbm.shape[0] // NW
    n_chunks = per_w // CHUNK
    w0 = wid * per_w
    cw = CHUNK * D

    lane_iota = lax.broadcasted_iota(jnp.int32, (LANES,), 0)

    def issue_in(c, b):
        pltpu.async_copy(unfair_hbm.at[pl.ds((w0 + c * CHUNK) * D, cw)],
                         buf_v[b], sem_in[b])

    def wait_in(b):
        pltpu.make_async_copy(unfair_hbm.at[pl.ds(0, cw)],
                              buf_v[b], sem_in[b]).wait()

    def issue_out(c, b):
        pltpu.async_copy(buf_v[b],
                         out_hbm.at[pl.ds((w0 + c * CHUNK) * D, cw)],
                         sem_out[b])

    def wait_out(b):
        pltpu.make_async_copy(buf_v[b], out_hbm.at[pl.ds(0, cw)],
                              sem_out[b]).wait()

    def or_tree(x):
        for sh in (1, 2, 4, 8):
            cnt_v[pl.ds(0, LANES)] = x
            x = x | plsc.load_gather(cnt_v, [(lane_iota + sh) & (LANES - 1)])
        return x

    def fixup(a, c):
        buf = buf_v[a]
        fid = fid_s[a]
        acc = fid[pl.ds(0, LANES)]
        for g in range(1, CHUNK // LANES):
            acc = acc | fid[pl.ds(g * LANES, LANES)]

        @pl.when(or_tree(acc)[0] != 0)
        def _dirty_chunk():
            def group_body(g, _):
                tg = g * LANES
                fid16 = fid[pl.ds(tg, LANES)]

                @pl.when(or_tree(fid16)[0] != 0)
                def _group():
                    pidx_v[pl.ds(0, LANES)] = (
                        w0 + c * CHUNK + tg + lane_iota) % L
                    cpf = pltpu.async_copy(
                        fair_hbm.at[fid.at[pl.ds(tg, LANES)]],
                        stf_v, sem_fix)
                    cpp = pltpu.async_copy(
                        pos_hbm.at[pidx_v.at[pl.ds(0, LANES)]],
                        stp_v, sem_fix)
                    cpf.wait()
                    cpp.wait()
                    for k in range(LANES):
                        @pl.when(fid16[k] != 0)
                        def _row(k=k):
                            base = (tg + k) * D
                            for d0 in range(0, D, LANES):
                                idx = base + d0 + lane_iota
                                vals = (stf_v[k, pl.ds(d0, LANES)]
                                        + stp_v[k, pl.ds(d0, LANES)])
                                plsc.store_scatter(buf, [idx], vals)

                return 0

            lax.fori_loop(0, CHUNK // LANES, group_body, 0)

    def issue_ids(c, b):
        pltpu.async_copy(ids_hbm.at[pl.ds(w0 + c * CHUNK, CHUNK)],
                         ids_s[b], sem_ids[b])

    def wait_ids(b):
        pltpu.make_async_copy(ids_hbm.at[pl.ds(0, CHUNK)],
                              ids_s[b], sem_ids[b]).wait()

    def issue_gather(b):
        for j in range(CHUNK // GW):
            pltpu.async_copy(tm_hbm.at[ids_s[b].at[pl.ds(j * GW, GW)]],
                             fid_s[b].at[pl.ds(j * GW, GW)], sem_g[b])

    def wait_gather(b):
        for j in range(CHUNK // GW):
            pltpu.make_async_copy(tm_hbm.at[ids_s[b].at[pl.ds(j * GW, GW)]],
                                  fid_s[b].at[pl.ds(j * GW, GW)],
                                  sem_g[b]).wait()

    issue_in(0, 0)
    issue_ids(0, 0)
    wait_ids(0)
    issue_gather(0)

    def pair_body(p, _):
        for par in range(2):
            c = p * 2 + par
            a, b = par, 1 - par

            @pl.when(c + 1 < n_chunks)
            def _prefetch():
                @pl.when(c > 0)
                def _():
                    wait_out(b)
                issue_in(c + 1, b)
                issue_ids(c + 1, b)

            wait_in(a)
            wait_gather(a)
            fixup(a, c)

            @pl.when(c + 1 < n_chunks)
            def _next_gather():
                wait_ids(b)
                issue_gather(b)

            issue_out(c, a)
        return 0

    lax.fori_loop(0, n_chunks // 2, pair_body, 0)

    wait_out(0)
    wait_out(1)


def kernel(input_ids, unfair_embeds, fair_table, pos_table, token_map):
    B, L = input_ids.shape
    D = unfair_embeds.shape[-1]
    N = B * L
    per_w = N // NW
    assert N % NW == 0 and per_w % CHUNK == 0 and per_w % GW == 0
    assert (per_w // CHUNK) % 2 == 0 and D % LANES == 0

    ids_flat = input_ids.reshape(N)
    unfair = unfair_embeds.reshape(N * D)
    pos_sl = pos_table[:L]

    mesh = plsc.VectorSubcoreMesh(core_axis_name="c", subcore_axis_name="s",
                                  num_cores=NC, num_subcores=NS)
    kfn = pl.kernel(
        functools.partial(_body, L, D),
        out_type=jax.ShapeDtypeStruct((N * D,), jnp.float32),
        mesh=mesh,
        scratch_types=[
            pltpu.VMEM((CHUNK,), jnp.int32),
            pltpu.VMEM((CHUNK,), jnp.int32),
            pltpu.VMEM((CHUNK,), jnp.int32),
            pltpu.VMEM((CHUNK,), jnp.int32),
            pltpu.VMEM((CHUNK * D,), jnp.float32),
            pltpu.VMEM((CHUNK * D,), jnp.float32),
            pltpu.VMEM((LANES, D), jnp.float32),
            pltpu.VMEM((LANES, D), jnp.float32),
            pltpu.VMEM((LANES,), jnp.int32),
            pltpu.VMEM((128,), jnp.int32),
            pltpu.SemaphoreType.DMA,
            pltpu.SemaphoreType.DMA,
            pltpu.SemaphoreType.DMA,
            pltpu.SemaphoreType.DMA,
            pltpu.SemaphoreType.DMA,
            pltpu.SemaphoreType.DMA,
            pltpu.SemaphoreType.DMA,
            pltpu.SemaphoreType.DMA,
            pltpu.SemaphoreType.DMA,
        ],
        compiler_params=pltpu.CompilerParams(needs_layout_passes=False),
    )
    out = kfn(ids_flat, unfair, fair_table, pos_sl, token_map)
    return out.reshape(B, L, D)


# speedup vs baseline: 1.0038x; 1.0038x over previous
import functools

import jax
import jax.numpy as jnp
from jax import lax
from jax.experimental import pallas as pl
from jax.experimental.pallas import tpu as pltpu
from jax.experimental.pallas import tpu_sc as plsc

NC = 2
NS = 16
LANES = 16
NW = NC * NS

CHUNK = 400
GW = 80


def _body(L, D, ids_hbm, unfair_hbm, fair_hbm, pos_hbm, tm_hbm,
          out_hbm,
          ids_s0, ids_s1, fid_s0, fid_s1, buf0, buf1,
          stf_v, stp_v, pidx_v, cnt_v,
          sem_in0, sem_in1, sem_out0, sem_out1,
          sem_ids0, sem_ids1, sem_g0, sem_g1, sem_fix):
    ids_s = (ids_s0, ids_s1)
    fid_s = (fid_s0, fid_s1)
    buf_v = (buf0, buf1)
    sem_in = (sem_in0, sem_in1)
    sem_out = (sem_out0, sem_out1)
    sem_ids = (sem_ids0, sem_ids1)
    sem_g = (sem_g0, sem_g1)

    wid = lax.axis_index("s") * NC + lax.axis_index("c")
    per_w = ids_hbm.shape[0] // NW
    n_chunks = per_w // CHUNK
    w0 = wid * per_w

    lane_iota = lax.broadcasted_iota(jnp.int32, (LANES,), 0)

    def issue_in(c, b):
        pltpu.async_copy(unfair_hbm.at[pl.ds(w0 + c * CHUNK, CHUNK)],
                         buf_v[b], sem_in[b])

    def wait_in(b):
        pltpu.make_async_copy(unfair_hbm.at[pl.ds(0, CHUNK)],
                              buf_v[b], sem_in[b]).wait()

    def issue_out(c, b):
        pltpu.async_copy(buf_v[b],
                         out_hbm.at[pl.ds(w0 + c * CHUNK, CHUNK)],
                         sem_out[b])

    def wait_out(b):
        pltpu.make_async_copy(buf_v[b], out_hbm.at[pl.ds(0, CHUNK)],
                              sem_out[b]).wait()

    def or_tree(x):
        for sh in (1, 2, 4, 8):
            cnt_v[pl.ds(0, LANES)] = x
            x = x | plsc.load_gather(cnt_v, [(lane_iota + sh) & (LANES - 1)])
        return x

    def fixup(a, c):
        buf = buf_v[a]
        fid = fid_s[a]
        acc = fid[pl.ds(0, LANES)]
        for g in range(1, CHUNK // LANES):
            acc = acc | fid[pl.ds(g * LANES, LANES)]

        @pl.when(or_tree(acc)[0] != 0)
        def _dirty_chunk():
            def group_body(g, _):
                tg = g * LANES
                fid16 = fid[pl.ds(tg, LANES)]

                @pl.when(or_tree(fid16)[0] != 0)
                def _group():
                    pidx_v[pl.ds(0, LANES)] = (
                        w0 + c * CHUNK + tg + lane_iota) % L
                    cpf = pltpu.async_copy(
                        fair_hbm.at[fid.at[pl.ds(tg, LANES)]],
                        stf_v, sem_fix)
                    cpp = pltpu.async_copy(
                        pos_hbm.at[pidx_v.at[pl.ds(0, LANES)]],
                        stp_v, sem_fix)
                    cpf.wait()
                    cpp.wait()
                    for k in range(LANES):
                        @pl.when(fid16[k] != 0)
                        def _row(k=k):
                            rowspl = jnp.full((LANES,), tg + k, jnp.int32)
                            for d0 in range(0, D, LANES):
                                vals = (stf_v[k, pl.ds(d0, LANES)]
                                        + stp_v[k, pl.ds(d0, LANES)])
                                plsc.store_scatter(
                                    buf, [rowspl, d0 + lane_iota], vals)

                return 0

            lax.fori_loop(0, CHUNK // LANES, group_body, 0)

    def issue_ids(c, b):
        pltpu.async_copy(ids_hbm.at[pl.ds(w0 + c * CHUNK, CHUNK)],
                         ids_s[b], sem_ids[b])

    def wait_ids(b):
        pltpu.make_async_copy(ids_hbm.at[pl.ds(0, CHUNK)],
                              ids_s[b], sem_ids[b]).wait()

    def issue_gather(b):
        for j in range(CHUNK // GW):
            pltpu.async_copy(tm_hbm.at[ids_s[b].at[pl.ds(j * GW, GW)]],
                             fid_s[b].at[pl.ds(j * GW, GW)], sem_g[b])

    def wait_gather(b):
        for j in range(CHUNK // GW):
            pltpu.make_async_copy(tm_hbm.at[ids_s[b].at[pl.ds(j * GW, GW)]],
                                  fid_s[b].at[pl.ds(j * GW, GW)],
                                  sem_g[b]).wait()

    issue_in(0, 0)
    issue_ids(0, 0)
    wait_ids(0)
    issue_gather(0)

    def pair_body(p, _):
        for par in range(2):
            c = p * 2 + par
            a, b = par, 1 - par

            @pl.when(c + 1 < n_chunks)
            def _prefetch():
                @pl.when(c > 0)
                def _():
                    wait_out(b)
                issue_in(c + 1, b)
                issue_ids(c + 1, b)

            wait_in(a)
            wait_gather(a)
            fixup(a, c)

            @pl.when(c + 1 < n_chunks)
            def _next_gather():
                wait_ids(b)
                issue_gather(b)

            issue_out(c, a)
        return 0

    lax.fori_loop(0, n_chunks // 2, pair_body, 0)

    wait_out(0)
    wait_out(1)


def kernel(input_ids, unfair_embeds, fair_table, pos_table, token_map):
    B, L = input_ids.shape
    D = unfair_embeds.shape[-1]
    N = B * L
    per_w = N // NW
    assert N % NW == 0 and per_w % CHUNK == 0 and per_w % GW == 0
    assert (per_w // CHUNK) % 2 == 0 and D % LANES == 0

    ids_flat = input_ids.reshape(N)
    unfair = unfair_embeds.reshape(N, D)
    pos_sl = pos_table[:L]

    mesh = plsc.VectorSubcoreMesh(core_axis_name="c", subcore_axis_name="s",
                                  num_cores=NC, num_subcores=NS)
    kfn = pl.kernel(
        functools.partial(_body, L, D),
        out_type=jax.ShapeDtypeStruct((N, D), jnp.float32),
        mesh=mesh,
        scratch_types=[
            pltpu.VMEM((CHUNK,), jnp.int32),
            pltpu.VMEM((CHUNK,), jnp.int32),
            pltpu.VMEM((CHUNK,), jnp.int32),
            pltpu.VMEM((CHUNK,), jnp.int32),
            pltpu.VMEM((CHUNK, D), jnp.float32),
            pltpu.VMEM((CHUNK, D), jnp.float32),
            pltpu.VMEM((LANES, D), jnp.float32),
            pltpu.VMEM((LANES, D), jnp.float32),
            pltpu.VMEM((LANES,), jnp.int32),
            pltpu.VMEM((128,), jnp.int32),
            pltpu.SemaphoreType.DMA,
            pltpu.SemaphoreType.DMA,
            pltpu.SemaphoreType.DMA,
            pltpu.SemaphoreType.DMA,
            pltpu.SemaphoreType.DMA,
            pltpu.SemaphoreType.DMA,
            pltpu.SemaphoreType.DMA,
            pltpu.SemaphoreType.DMA,
            pltpu.SemaphoreType.DMA,
        ],
        compiler_params=pltpu.CompilerParams(needs_layout_passes=False),
    )
    out = kfn(ids_flat, unfair, fair_table, pos_sl, token_map)
    return out.reshape(B, L, D)


# speedup vs baseline: 2.5334x; 2.5239x over previous
import functools

import jax
import jax.numpy as jnp
from jax import lax
from jax.experimental import pallas as pl
from jax.experimental.pallas import tpu as pltpu
from jax.experimental.pallas import tpu_sc as plsc

NC = 2
NS = 16
LANES = 16
NW = NC * NS

CHUNK = 320
GSEG = [(o, min(128, CHUNK - o)) for o in range(0, CHUNK, 128)]


def _body(L, ids_hbm, unfair_hbm, fair_hbm, pos_hbm, tm_hbm,
          out_hbm,
          ids_v0, ids_v1, ids_v2, ids_v3,
          fid_v0, fid_v1, fid_v2, fid_v3, buf_v0, buf_v1,
          fair_v, pos_v, cnt_v,
          sem_in0, sem_in1, sem_out0, sem_out1,
          sem_ids0, sem_ids1, sem_ids2, sem_ids3,
          sem_g0, sem_g1, sem_g2, sem_g3):
    ids_v = (ids_v0, ids_v1, ids_v2, ids_v3)
    fid_v = (fid_v0, fid_v1, fid_v2, fid_v3)
    buf_v = (buf_v0, buf_v1)
    sem_in = (sem_in0, sem_in1)
    sem_out = (sem_out0, sem_out1)
    sem_ids = (sem_ids0, sem_ids1, sem_ids2, sem_ids3)
    sem_g = (sem_g0, sem_g1, sem_g2, sem_g3)

    wid = lax.axis_index("s") * NC + lax.axis_index("c")
    per_w = unfair_hbm.shape[0] // NW
    n_chunks = per_w // CHUNK

    pltpu.sync_copy(fair_hbm, fair_v)
    pltpu.sync_copy(pos_hbm, pos_v)

    lane_iota = lax.broadcasted_iota(jnp.int32, (LANES,), 0)

    def r0_of(c):
        return wid * per_w + c * CHUNK

    def issue_in(c, b):
        pltpu.async_copy(unfair_hbm.at[pl.ds(r0_of(c), CHUNK)],
                         buf_v[b], sem_in[b])

    def wait_in(b):
        pltpu.make_async_copy(unfair_hbm.at[pl.ds(0, CHUNK)],
                              buf_v[b], sem_in[b]).wait()

    def issue_out(c, b):
        pltpu.async_copy(buf_v[b], out_hbm.at[pl.ds(r0_of(c), CHUNK)],
                         sem_out[b])

    def wait_out(b):
        pltpu.make_async_copy(buf_v[b], out_hbm.at[pl.ds(0, CHUNK)],
                              sem_out[b]).wait()

    def issue_ids(c, b):
        pltpu.async_copy(ids_hbm.at[pl.ds(r0_of(c), CHUNK)],
                         ids_v[b], sem_ids[b])

    def wait_ids(b):
        pltpu.make_async_copy(ids_hbm.at[pl.ds(0, CHUNK)],
                              ids_v[b], sem_ids[b]).wait()

    def issue_gather(b):
        for o, w in GSEG:
            pltpu.async_copy(tm_hbm.at[ids_v[b].at[pl.ds(o, w)]],
                             fid_v[b].at[pl.ds(o, w)], sem_g[b])

    def wait_gather(b):
        for o, w in GSEG:
            pltpu.make_async_copy(tm_hbm.at[ids_v[b].at[pl.ds(o, w)]],
                                  fid_v[b].at[pl.ds(o, w)], sem_g[b]).wait()

    def or_tree(x):
        for sh in (1, 2, 4, 8):
            cnt_v[pl.ds(0, LANES)] = x
            x = x | plsc.load_gather(cnt_v, [(lane_iota + sh) & (LANES - 1)])
        return x

    def fixup(gi, a, r0):
        buf = buf_v[a]
        fid = fid_v[gi]
        acc = fid[pl.ds(0, LANES)]
        for g in range(1, CHUNK // LANES):
            acc = acc | fid[pl.ds(g * LANES, LANES)]

        @pl.when(or_tree(acc)[0] != 0)
        def _dirty_chunk():
            def group_body(g, _):
                fid16 = fid[pl.ds(g * LANES, LANES)]
                mask = fid16 != 0

                @pl.when(or_tree(fid16)[0] != 0)
                def _group():
                    row16 = g * LANES + lane_iota
                    pos16 = (r0 + row16) % L
                    for c0 in range(buf.shape[-1]):
                        col = jnp.full((LANES,), c0, jnp.int32)
                        vals = (plsc.load_gather(fair_v, [fid16, col])
                                + plsc.load_gather(pos_v, [pos16, col]))
                        plsc.store_scatter(buf, [row16, col], vals,
                                           mask=mask)

                return 0

            lax.fori_loop(0, CHUNK // LANES, group_body, 0)

    issue_in(0, 0)
    issue_ids(0, 0)
    issue_ids(1, 1)
    wait_ids(0)
    issue_gather(0)
    wait_ids(1)
    issue_gather(1)

    def quad_body(p, _):
        for par in range(4):
            c = p * 4 + par
            a, b = par % 2, 1 - par % 2
            gi = par
            gn = (par + 2) % 4

            @pl.when(c + 1 < n_chunks)
            def _prefetch():
                @pl.when(c > 0)
                def _():
                    wait_out(b)
                issue_in(c + 1, b)

            @pl.when(c + 2 < n_chunks)
            def _prefetch_ids():
                issue_ids(c + 2, gn)

            wait_in(a)
            wait_gather(gi)
            fixup(gi, a, r0_of(c))

            @pl.when(c + 2 < n_chunks)
            def _next_gather():
                wait_ids(gn)
                issue_gather(gn)

            issue_out(c, a)
        return 0

    lax.fori_loop(0, n_chunks // 4, quad_body, 0)

    wait_out(0)
    wait_out(1)


def kernel(input_ids, unfair_embeds, fair_table, pos_table, token_map):
    B, L = input_ids.shape
    D = unfair_embeds.shape[-1]
    N = B * L
    assert N % (NW * CHUNK) == 0 and (N // (NW * CHUNK)) % 4 == 0 and D == 128

    ids_flat = input_ids.reshape(N)
    unfair = unfair_embeds.reshape(N, D)
    pos_sl = pos_table[:L]

    mesh = plsc.VectorSubcoreMesh(core_axis_name="c", subcore_axis_name="s",
                                  num_cores=NC, num_subcores=NS)
    kfn = pl.kernel(
        functools.partial(_body, L),
        out_type=jax.ShapeDtypeStruct((N, D), jnp.float32),
        mesh=mesh,
        scratch_types=[
            pltpu.VMEM((CHUNK,), jnp.int32),
            pltpu.VMEM((CHUNK,), jnp.int32),
            pltpu.VMEM((CHUNK,), jnp.int32),
            pltpu.VMEM((CHUNK,), jnp.int32),
            pltpu.VMEM((CHUNK,), jnp.int32),
            pltpu.VMEM((CHUNK,), jnp.int32),
            pltpu.VMEM((CHUNK,), jnp.int32),
            pltpu.VMEM((CHUNK,), jnp.int32),
            pltpu.VMEM((CHUNK, D), jnp.float32),
            pltpu.VMEM((CHUNK, D), jnp.float32),
            pltpu.VMEM((fair_table.shape[0], D), jnp.float32),
            pltpu.VMEM((L, D), jnp.float32),
            pltpu.VMEM((128,), jnp.int32),
        ] + [pltpu.SemaphoreType.DMA] * 12,
        compiler_params=pltpu.CompilerParams(needs_layout_passes=False),
    )
    out = kfn(ids_flat, unfair, fair_table, pos_sl, token_map)
    return out.reshape(B, L, D)
